# baseline (speedup 1.0000x reference)
import jax
import jax.numpy as jnp
import numpy as np
from jax.experimental import pallas as pl

B, L, V, D, TD, NH, TOPK, TEMP = 1024, 50, 100000, 64, 512, 4, 20, 0.07


def _ln(x, eps=1e-12):
    mu = jnp.mean(x, axis=-1, keepdims=True)
    var = jnp.mean((x - mu) ** 2, axis=-1, keepdims=True)
    return (x - mu) / jnp.sqrt(var + eps)


def _mod_proj(x, p):
    h = jax.nn.gelu(jnp.dot(x, p['w1']) + p['b1'])
    h = jnp.dot(h, p['w2']) + p['b2']
    out = p['alpha'] * h + jnp.dot(x, p['rw']) + p['rb']
    return _ln(out, 1e-5)


def _mha(q_in, kv_in, p, mask):
    Bq, Lq, Dm = q_in.shape
    hd = Dm // NH
    def split(t):
        return t.reshape(Bq, -1, NH, hd).transpose(0, 2, 1, 3)
    q = split(jnp.dot(q_in, p['wq']) + p['bq'])
    k = split(jnp.dot(kv_in, p['wk']) + p['bk'])
    v = split(jnp.dot(kv_in, p['wv']) + p['bv'])
    logits = jnp.einsum('bhqd,bhkd->bhqk', q, k) / np.sqrt(hd)
    logits = jnp.where(mask[:, None, None, :], logits, -1e9)
    a = jax.nn.softmax(logits, axis=-1)
    o = jnp.einsum('bhqk,bhkd->bhqd', a, v)
    o = o.transpose(0, 2, 1, 3).reshape(Bq, Lq, Dm)
    return jnp.dot(o, p['wo']) + p['bo']


def _self_attend(seq, maskf, p):
    Q = jnp.dot(seq, p['wq']) + p['bq']
    K = jnp.dot(seq, p['wk']) + p['bk']
    Vv = jnp.dot(seq, p['wv']) + p['bv']
    attn = jnp.einsum('bqd,bkd->bqk', Q, K) / np.sqrt(D)
    attn = jnp.where(maskf[:, None, :] > 0, attn, -1e9)
    a = jax.nn.softmax(attn, axis=-1)
    out = jnp.einsum('bqk,bkd->bqd', a, Vv)
    pref = jnp.sum(out * maskf[..., None], axis=1) / jnp.sum(maskf, axis=1, keepdims=True)
    return jnp.dot(pref, p['wo']) + p['bo']


def _passthrough(x_ref, o_ref):
    o_ref[...] = x_ref[...]


def kernel(params, text_feat, image_feat, hist, seq_mask):
    maskf = seq_mask.astype(jnp.float32)
    item_seq = params['item_table'][hist]
    text_seq = _mod_proj(text_feat[hist], params['tp'])
    vis_seq = _mod_proj(image_feat[hist], params['vp'])
    pos = params['pos_table'][:L][None, :, :]
    item_seq = _ln(item_seq + pos)
    text_seq = _ln(text_seq + pos)
    vis_seq = _ln(vis_seq + pos)
    den = params['den']
    tr = _mha(text_seq, vis_seq, den['t2v'], seq_mask)
    vr = _mha(vis_seq, text_seq, den['v2t'], seq_mask)
    g = jax.nn.softmax(jnp.dot(jax.nn.gelu(jnp.dot(item_seq, den['g1']) + den['gb1']), den['g2']) + den['gb2'], axis=-1)
    text_out = _ln(text_seq + den['res'] * (g[..., 0:1] * tr), 1e-5)
    vis_out = _ln(vis_seq + den['res'] * (g[..., 1:2] * vr), 1e-5)
    ip = _self_attend(item_seq, maskf, params['up']['item'])
    tp_ = _self_attend(text_out, maskf, params['up']['text'])
    vp_ = _self_attend(vis_out, maskf, params['up']['visual'])
    cat = jnp.concatenate([ip, tp_, vp_], axis=-1)
    rt = params['rt']
    h = jnp.dot(cat, rt['w1']) + rt['b1']
    h = h / jnp.maximum(jnp.linalg.norm(h, axis=1, keepdims=True), 1e-12)
    w = jax.nn.softmax((jnp.dot(jax.nn.relu(h), rt['w2']) + rt['b2']) / TEMP, axis=1)
    user = w[:, 0:1] * ip + w[:, 1:2] * tp_ + w[:, 2:3] * vp_
    scores = jnp.dot(user, params['item_table'][:V].T)
    vals, idx = jax.lax.top_k(scores, TOPK)
    vals = pl.pallas_call(
        _passthrough,
        out_shape=jax.ShapeDtypeStruct(vals.shape, vals.dtype),
    )(vals)
    return vals, idx


# speedup vs baseline: 2.0108x; 2.0108x over previous
import functools

import jax
import jax.numpy as jnp
import numpy as np
from jax import lax
from jax.experimental import pallas as pl
from jax.experimental.pallas import tpu as pltpu
from jax.experimental.pallas import tpu_sc as plsc

B, L, V, D, TD, NH, TOPK, TEMP = 1024, 50, 100000, 64, 512, 4, 20, 0.07
VP = 102400
CW = 128
NCH = VP // CW
TN = 2048
BM = 256
RB = 128
NEG = -1.0e30
NEGF = -3.0e38
BIGI = 2 ** 30


def _ln(x, eps=1e-12):
    mu = jnp.mean(x, axis=-1, keepdims=True)
    var = jnp.mean((x - mu) ** 2, axis=-1, keepdims=True)
    return (x - mu) / jnp.sqrt(var + eps)


def _mod_proj(x, p):
    h = jax.nn.gelu(jnp.dot(x, p['w1']) + p['b1'])
    h = jnp.dot(h, p['w2']) + p['b2']
    out = p['alpha'] * h + jnp.dot(x, p['rw']) + p['rb']
    return _ln(out, 1e-5)


def _mha(q_in, kv_in, p, mask):
    Bq, Lq, Dm = q_in.shape
    hd = Dm // NH
    def split(t):
        return t.reshape(Bq, -1, NH, hd).transpose(0, 2, 1, 3)
    q = split(jnp.dot(q_in, p['wq']) + p['bq'])
    k = split(jnp.dot(kv_in, p['wk']) + p['bk'])
    v = split(jnp.dot(kv_in, p['wv']) + p['bv'])
    logits = jnp.einsum('bhqd,bhkd->bhqk', q, k) / np.sqrt(hd)
    logits = jnp.where(mask[:, None, None, :], logits, -1e9)
    a = jax.nn.softmax(logits, axis=-1)
    o = jnp.einsum('bhqk,bhkd->bhqd', a, v)
    o = o.transpose(0, 2, 1, 3).reshape(Bq, Lq, Dm)
    return jnp.dot(o, p['wo']) + p['bo']


def _self_attend(seq, maskf, p):
    Q = jnp.dot(seq, p['wq']) + p['bq']
    K = jnp.dot(seq, p['wk']) + p['bk']
    Vv = jnp.dot(seq, p['wv']) + p['bv']
    attn = jnp.einsum('bqd,bkd->bqk', Q, K) / np.sqrt(D)
    attn = jnp.where(maskf[:, None, :] > 0, attn, -1e9)
    a = jax.nn.softmax(attn, axis=-1)
    out = jnp.einsum('bqk,bkd->bqd', a, Vv)
    pref = jnp.sum(out * maskf[..., None], axis=1) / jnp.sum(maskf, axis=1, keepdims=True)
    return jnp.dot(pref, p['wo']) + p['bo']


def _encoder(params, text_feat, image_feat, hist, seq_mask):
    maskf = seq_mask.astype(jnp.float32)
    item_seq = params['item_table'][hist]
    text_seq = _mod_proj(text_feat[hist], params['tp'])
    vis_seq = _mod_proj(image_feat[hist], params['vp'])
    pos = params['pos_table'][:L][None, :, :]
    item_seq = _ln(item_seq + pos)
    text_seq = _ln(text_seq + pos)
    vis_seq = _ln(vis_seq + pos)
    den = params['den']
    tr = _mha(text_seq, vis_seq, den['t2v'], seq_mask)
    vr = _mha(vis_seq, text_seq, den['v2t'], seq_mask)
    g = jax.nn.softmax(jnp.dot(jax.nn.gelu(jnp.dot(item_seq, den['g1']) + den['gb1']), den['g2']) + den['gb2'], axis=-1)
    text_out = _ln(text_seq + den['res'] * (g[..., 0:1] * tr), 1e-5)
    vis_out = _ln(vis_seq + den['res'] * (g[..., 1:2] * vr), 1e-5)
    ip = _self_attend(item_seq, maskf, params['up']['item'])
    tp_ = _self_attend(text_out, maskf, params['up']['text'])
    vp_ = _self_attend(vis_out, maskf, params['up']['visual'])
    cat = jnp.concatenate([ip, tp_, vp_], axis=-1)
    rt = params['rt']
    h = jnp.dot(cat, rt['w1']) + rt['b1']
    h = h / jnp.maximum(jnp.linalg.norm(h, axis=1, keepdims=True), 1e-12)
    w = jax.nn.softmax((jnp.dot(jax.nn.relu(h), rt['w2']) + rt['b2']) / TEMP, axis=1)
    user = w[:, 0:1] * ip + w[:, 1:2] * tp_ + w[:, 2:3] * vp_
    return user


def _scores_body(u_ref, t_ref, s_ref, m_ref):
    j = pl.program_id(0)
    s = lax.dot_general(u_ref[...], t_ref[...], (((1,), (1,)), ((), ())))
    col = j * TN + lax.broadcasted_iota(jnp.int32, (BM, TN), 1)
    s = jnp.where(col < V, s, NEG)
    s_ref[...] = s
    m_ref[0] = jnp.max(s.reshape(BM, TN // CW, CW), axis=2)


def _scores_topchunks(user, tpad):
    return pl.pallas_call(
        _scores_body,
        grid=(VP // TN, B // BM),
        in_specs=[
            pl.BlockSpec((BM, D), lambda j, i: (i, 0)),
            pl.BlockSpec((TN, D), lambda j, i: (j, 0)),
        ],
        out_specs=[
            pl.BlockSpec((BM, TN), lambda j, i: (i, j)),
            pl.BlockSpec((1, BM, TN // CW), lambda j, i: (j, i, 0)),
        ],
        out_shape=[
            jax.ShapeDtypeStruct((B, VP), jnp.float32),
            jax.ShapeDtypeStruct((VP // TN, B, TN // CW), jnp.float32),
        ],
    )(user, tpad)


def _l1_body(m_ref, o_ref):
    i = pl.program_id(0)
    x = m_ref[...]
    iota_c = lax.broadcasted_iota(jnp.int32, (RB, NCH), 1)
    rowbase = (i * RB + lax.broadcasted_iota(jnp.int32, (RB, 1), 0)) * NCH
    cols = []
    for _ in range(TOPK):
        m = jnp.max(x, axis=1, keepdims=True)
        sel = jnp.min(jnp.where(x == m, iota_c, BIGI), axis=1, keepdims=True)
        cols.append(sel + rowbase)
        x = jnp.where(iota_c == sel, NEGF, x)
    cols.append(jnp.zeros((RB, 128 - TOPK), jnp.int32))
    o_ref[...] = jnp.concatenate(cols, axis=1)


def _top_chunk_ids(cmax):
    out = pl.pallas_call(
        _l1_body,
        grid=(B // RB,),
        in_specs=[pl.BlockSpec((RB, NCH), lambda i: (i, 0))],
        out_specs=pl.BlockSpec((RB, 128), lambda i: (i, 0)),
        out_shape=jax.ShapeDtypeStruct((B, 128), jnp.int32),
    )(cmax)
    return out[:, :TOPK]


_NW = 32
_GROUPS = B * TOPK // 128
_GPW = _GROUPS // _NW


@functools.lru_cache(maxsize=None)
def _build_gather_chunks():
    @functools.partial(
        pl.kernel,
        mesh=plsc.VectorSubcoreMesh(core_axis_name="c", subcore_axis_name="s"),
        out_type=jax.ShapeDtypeStruct((B * TOPK, CW), jnp.float32),
        scratch_types=[
            pltpu.VMEM((128,), jnp.int32),
            pltpu.VMEM((128, CW), jnp.float32),
            pltpu.SemaphoreType.DMA,
        ],
    )
    def _gather_chunks(scores_hbm, idx_hbm, out_hbm, idx_v, rows_v, sem):
        wid = lax.axis_index("s") * 2 + lax.axis_index("c")
        for j in range(_GPW):
            r = wid * _GPW + j
            pltpu.sync_copy(idx_hbm.at[r], idx_v)
            pltpu.async_copy(scores_hbm.at[idx_v], rows_v, sem).wait()
            pltpu.sync_copy(rows_v, out_hbm.at[pl.ds(r * 128, 128)])

    return _gather_chunks


def _l2_body(c_ref, f_ref, v_ref, i_ref):
    i = pl.program_id(0)
    x = c_ref[...]
    rowbase = (i * RB + lax.broadcasted_iota(jnp.int32, (RB, 1), 0)) * NCH
    cid = f_ref[...] - rowbase
    idxarr = cid[:, :, None] * CW + lax.broadcasted_iota(
        jnp.int32, (RB, TOPK, CW), 2)
    vs, ids = [], []
    for _ in range(TOPK):
        m = jnp.max(jnp.max(x, axis=2), axis=1)
        m3 = m[:, None, None]
        hit = x == m3
        sel = jnp.min(jnp.min(jnp.where(hit, idxarr, BIGI), axis=2), axis=1)
        vs.append(m[:, None])
        ids.append(sel[:, None])
        x = jnp.where(hit & (idxarr == sel[:, None, None]), NEGF, x)
    v_ref[...] = jnp.concatenate(vs, axis=1)
    i_ref[...] = jnp.concatenate(ids, axis=1)


def _topk_of_candidates(cand, fidx):
    return pl.pallas_call(
        _l2_body,
        grid=(B // RB,),
        in_specs=[
            pl.BlockSpec((RB, TOPK, CW), lambda i: (i, 0, 0)),
            pl.BlockSpec((RB, TOPK), lambda i: (i, 0)),
        ],
        out_specs=[
            pl.BlockSpec((RB, TOPK), lambda i: (i, 0)),
            pl.BlockSpec((RB, TOPK), lambda i: (i, 0)),
        ],
        out_shape=[
            jax.ShapeDtypeStruct((B, TOPK), jnp.float32),
            jax.ShapeDtypeStruct((B, TOPK), jnp.int32),
        ],
    )(cand, fidx)


def kernel(params, text_feat, image_feat, hist, seq_mask):
    user = _encoder(params, text_feat, image_feat, hist, seq_mask)
    tpad = jnp.pad(params['item_table'][:V], ((0, VP - V), (0, 0)))
    scores, cmax3 = _scores_topchunks(user, tpad)
    cmax = cmax3.transpose(1, 0, 2).reshape(B, NCH)
    fidx = _top_chunk_ids(cmax)
    cand = _build_gather_chunks()(scores.reshape(B * NCH, CW),
                                  fidx.reshape(_GROUPS, 128))
    vals, idx = _topk_of_candidates(cand.reshape(B, TOPK, CW), fidx)
    return vals, idx
